# Initial kernel scaffold; baseline (speedup 1.0000x reference)
import jax
import jax.numpy as jnp
from jax.experimental import pallas as pl


def kernel(x, feature_types, W1, b1, W2, b2, W3, b3, type_emb, Wt, bt, We, be, Wo, bo):
    raise NotImplementedError("write your pallas kernel here")


# speedup vs baseline: 2.8770x; 2.8770x over previous
import functools

import jax
import jax.numpy as jnp
from jax.experimental import pallas as pl
from jax.experimental.pallas import tpu as pltpu

B, S, D = 2, 2048, 1024
H = 2048
E = 8
GH = 512
N = B * S
NEG = -1e30

GBLK = 1024
TBLK = 2048
HBLK = 1024
OBLK = 1024


def _gate_body(x_ref, ft_ref, w1_ref, b1_ref, w2_ref, b2_ref, w3_ref, b3_ref,
               temb_ref, wt_ref, bt_ref, g_ref):
    xb = x_ref[...]
    h = jnp.maximum(jnp.dot(xb, w1_ref[...],
                            preferred_element_type=jnp.float32) + b1_ref[...], 0.0)
    h = jnp.maximum(jnp.dot(h, w2_ref[...],
                            preferred_element_type=jnp.float32) + b2_ref[...], 0.0)
    gl = jnp.dot(h, w3_ref[...], preferred_element_type=jnp.float32) + b3_ref[...]

    tlt = jnp.dot(temb_ref[...], wt_ref[...],
                  preferred_element_type=jnp.float32) + bt_ref[...]
    ft = ft_ref[...]
    for c in range(3):
        gl = gl + jnp.where(ft == c, 1.0, 0.0) * tlt[c:c + 1, :]

    lane = jax.lax.broadcasted_iota(jnp.int32, (GBLK, E), 1)
    m1 = jnp.max(gl, axis=-1, keepdims=True)
    i1 = jnp.min(jnp.where(gl == m1, lane, E), axis=-1, keepdims=True)
    gl2 = jnp.where(lane == i1, NEG, gl)
    m2 = jnp.max(gl2, axis=-1, keepdims=True)
    i2 = jnp.min(jnp.where(gl2 == m2, lane, E), axis=-1, keepdims=True)
    e2 = jnp.exp(m2 - m1)
    wa = 1.0 / (1.0 + e2)
    wb = e2 / (1.0 + e2)
    g_ref[...] = (jnp.where(lane == i1, wa, 0.0)
                  + jnp.where(lane == i2, wb, 0.0))


def _expert_body(x_ref, g_ref, we_ref, be_ref, hid_ref):
    e = pl.program_id(2)
    he = jnp.maximum(jnp.dot(x_ref[...], we_ref[0],
                             preferred_element_type=jnp.float32)
                     + be_ref[0], 0.0)
    lane = jax.lax.broadcasted_iota(jnp.int32, (TBLK, E), 1)
    ge = jnp.sum(jnp.where(lane == e, g_ref[...], 0.0), axis=1, keepdims=True)
    val = ge * he

    @pl.when(e == 0)
    def _():
        hid_ref[...] = val

    @pl.when(e > 0)
    def _():
        hid_ref[...] = hid_ref[...] + val


def _proj_body(hid_ref, wo_ref, bo_ref, out_ref):
    out_ref[...] = jnp.dot(hid_ref[...], wo_ref[...],
                           preferred_element_type=jnp.float32) + bo_ref[...]


def _full(shape):
    return pl.BlockSpec(shape, lambda *_: tuple(0 for _ in shape))


@jax.jit
def _run(x2, ft2, W1, b1, W2, b2, W3, b3, type_emb, Wt, bt, We, be, Wo, bo):
    g = pl.pallas_call(
        _gate_body,
        grid=(N // GBLK,),
        in_specs=[
            pl.BlockSpec((GBLK, D), lambda i: (i, 0)),
            pl.BlockSpec((GBLK, 1), lambda i: (i, 0)),
            _full((D, GH)), _full((1, GH)),
            _full((GH, GH // 2)), _full((1, GH // 2)),
            _full((GH // 2, E)), _full((1, E)),
            _full((3, GH // 4)), _full((GH // 4, E)), _full((1, E)),
        ],
        out_specs=pl.BlockSpec((GBLK, E), lambda i: (i, 0)),
        out_shape=jax.ShapeDtypeStruct((N, E), jnp.float32),
    )(x2, ft2, W1, b1, W2, b2, W3, b3, type_emb, Wt, bt)

    hid = pl.pallas_call(
        _expert_body,
        grid=(H // HBLK, N // TBLK, E),
        in_specs=[
            pl.BlockSpec((TBLK, D), lambda hb, tb, e: (tb, 0)),
            pl.BlockSpec((TBLK, E), lambda hb, tb, e: (tb, 0)),
            pl.BlockSpec((1, D, HBLK), lambda hb, tb, e: (e, 0, hb)),
            pl.BlockSpec((1, 1, HBLK), lambda hb, tb, e: (e, 0, hb)),
        ],
        out_specs=pl.BlockSpec((TBLK, HBLK), lambda hb, tb, e: (tb, hb)),
        out_shape=jax.ShapeDtypeStruct((N, H), jnp.float32),
        compiler_params=pltpu.CompilerParams(
            dimension_semantics=("arbitrary", "arbitrary", "arbitrary"),
        ),
    )(x2, g, We, be.reshape(E, 1, H))

    out = pl.pallas_call(
        _proj_body,
        grid=(N // OBLK,),
        in_specs=[
            pl.BlockSpec((OBLK, H), lambda i: (i, 0)),
            _full((H, D)), _full((1, D)),
        ],
        out_specs=pl.BlockSpec((OBLK, D), lambda i: (i, 0)),
        out_shape=jax.ShapeDtypeStruct((N, D), jnp.float32),
    )(hid, Wo, bo)
    return out


def kernel(x, feature_types, W1, b1, W2, b2, W3, b3, type_emb, Wt, bt, We, be, Wo, bo):
    x2 = x.reshape(N, D)
    ft2 = feature_types.reshape(N, 1).astype(jnp.int32)
    out = _run(x2, ft2, W1, b1.reshape(1, GH), W2, b2.reshape(1, GH // 2),
               W3, b3.reshape(1, E), type_emb, Wt, bt.reshape(1, E),
               We, be, Wo, bo.reshape(1, D))
    return out.reshape(B, S, D)
